# baseline (speedup 1.0000x reference)
import jax
import jax.numpy as jnp
from jax import lax
from jax.experimental import pallas as pl
from jax.experimental.pallas import tpu as pltpu
from jax.experimental.pallas import tpu_sc as plsc

B = 16384
L = 50
D = 64
N = B * L

NW = 32
PER_W = N // NW
W = 256
CH = PER_W // W


def kernel(indices, table):
    tablep = jnp.pad(table, ((0, 0), (0, 128 - D)))
    idx_flat = indices.reshape(1, N).astype(jnp.int32)

    mesh = plsc.VectorSubcoreMesh(core_axis_name="core", subcore_axis_name="subcore")

    @pl.kernel(
        out_type=jax.ShapeDtypeStruct((N, D), table.dtype),
        mesh=mesh,
        scratch_types=[
            pltpu.VMEM((PER_W,), jnp.int32),
            pltpu.VMEM((2, W, 128), jnp.float32),
            pltpu.SemaphoreType.DMA,
            pltpu.SemaphoreType.DMA,
            pltpu.SemaphoreType.DMA,
        ],
        compiler_params=pltpu.CompilerParams(use_tc_tiling_on_sc=False),
    )
    def gather_kernel(table_hbm, idx_hbm, out_hbm, idx_v, rows_v, gsem, wsem, isem):
        wid = lax.axis_index("subcore") * 2 + lax.axis_index("core")
        base = wid * PER_W
        pltpu.async_copy(idx_hbm.at[0, pl.ds(base, PER_W)], idx_v, isem).wait()

        @pl.loop(0, CH, step=2)
        def _(g):
            ga = pltpu.async_copy(
                table_hbm.at[idx_v.at[pl.ds(g * W, W)]], rows_v.at[0], gsem
            )
            gb = pltpu.async_copy(
                table_hbm.at[idx_v.at[pl.ds((g + 1) * W, W)]], rows_v.at[1], gsem
            )
            ga.wait()
            wa = pltpu.async_copy(
                rows_v.at[0, :, pl.ds(0, D)],
                out_hbm.at[pl.ds(base + g * W, W)],
                wsem,
            )
            gb.wait()
            wb = pltpu.async_copy(
                rows_v.at[1, :, pl.ds(0, D)],
                out_hbm.at[pl.ds(base + (g + 1) * W, W)],
                wsem,
            )
            wa.wait()
            wb.wait()

    out = gather_kernel(tablep, idx_flat)
    return out.reshape(B, L, D)
